# baseline (speedup 1.0000x reference)
import functools
import math

import jax
import jax.numpy as jnp
from jax import lax
from jax.experimental import pallas as pl
from jax.experimental.pallas import tpu as pltpu
from jax.experimental.pallas import tpu_sc as plsc

_VOCAB = 100000
_EMBED = 128
_BATCH = 4096
_SEQ = 200
_SCALE = math.sqrt(float(_EMBED))

_NC = 2
_NS = 16
_NW = _NC * _NS
_TOKENS = _BATCH * _SEQ
_PER_W = _TOKENS // _NW
_C = 128
_NCHUNK = _PER_W // _C
_NBUF = 4
_NROUND = _NCHUNK // _NBUF


@functools.partial(
    pl.kernel,
    mesh=plsc.VectorSubcoreMesh(core_axis_name="c", subcore_axis_name="s"),
    out_type=jax.ShapeDtypeStruct((_NW, _NCHUNK, _C, _EMBED), jnp.float32),
    scratch_types=[
        pltpu.VMEM((_NCHUNK, _C), jnp.int32),
        pltpu.VMEM((_C, _EMBED), jnp.float32),
        pltpu.VMEM((_C, _EMBED), jnp.float32),
        pltpu.VMEM((_C, _EMBED), jnp.float32),
        pltpu.VMEM((_C, _EMBED), jnp.float32),
        pltpu.SemaphoreType.DMA,
        pltpu.SemaphoreType.DMA,
        pltpu.SemaphoreType.DMA,
        pltpu.SemaphoreType.DMA,
        pltpu.SemaphoreType.DMA,
        pltpu.SemaphoreType.DMA,
        pltpu.SemaphoreType.DMA,
        pltpu.SemaphoreType.DMA,
    ],
)
def _embed(table_hbm, idx_hbm, out_hbm, idx_v,
           r0, r1, r2, r3, g0, g1, g2, g3, s0, s1, s2, s3):
    wid = lax.axis_index("s") * _NC + lax.axis_index("c")
    pltpu.sync_copy(idx_hbm.at[wid], idx_v)

    bufs = (r0, r1, r2, r3)
    gsems = (g0, g1, g2, g3)
    ssems = (s0, s1, s2, s3)

    def start_gather(g, b):
        pltpu.make_async_copy(table_hbm.at[idx_v.at[g]], bufs[b], gsems[b]).start()

    def wait_gather(b):
        pltpu.make_async_copy(table_hbm.at[idx_v.at[0]], bufs[b], gsems[b]).wait()

    def start_scatter(g, b):
        pltpu.make_async_copy(bufs[b], out_hbm.at[wid, g], ssems[b]).start()

    def wait_scatter(b):
        pltpu.make_async_copy(bufs[b], out_hbm.at[wid, 0], ssems[b]).wait()

    def scale_buf(b):
        pass


    start_gather(0, 0)
    start_gather(1, 1)
    for b in (0, 1):
        wait_gather(b)
        scale_buf(b)
        start_scatter(b, b)
        start_gather(b + 2, b + 2)
    for b in (2, 3):
        wait_gather(b)
        scale_buf(b)
        start_scatter(b, b)
        wait_scatter(b - 2)
        start_gather(4 + b - 2, b - 2)

    def step(i, carry):
        for b in range(_NBUF):
            wait_gather(b)
            scale_buf(b)
            start_scatter(_NBUF * (i - 1) + b, b)
            b2 = (b + 2) % _NBUF
            wait_scatter(b2)
            if b2 >= 2:
                start_gather(_NBUF * (i - 1) + b2, b2)
            else:
                start_gather(_NBUF * i + b2, b2)
        return carry

    lax.fori_loop(2, _NROUND, step, 0)

    for b in (0, 1):
        wait_gather(b)
        scale_buf(b)
        start_scatter(_NCHUNK - _NBUF + b, b)
        wait_scatter(b + 2)
        start_gather(_NCHUNK - 2 + b, b + 2)
    for b in (2, 3):
        wait_gather(b)
        scale_buf(b)
        start_scatter(_NCHUNK - 2 + b - 2, b)
    for b in range(_NBUF):
        wait_scatter(b)


def _scale_body(w_ref, o_ref):
    o_ref[...] = w_ref[...] * _SCALE


def _scale_table(weight):
    blk = 2000
    return pl.pallas_call(
        _scale_body,
        grid=(_VOCAB // blk,),
        in_specs=[pl.BlockSpec((blk, _EMBED), lambda i: (i, 0))],
        out_specs=pl.BlockSpec((blk, _EMBED), lambda i: (i, 0)),
        out_shape=jax.ShapeDtypeStruct((_VOCAB, _EMBED), jnp.float32),
    )(weight)


def kernel(tokens, embedding_weight):
    idx = tokens.astype(jnp.int32).reshape(_NW, _NCHUNK, _C)
    out = _embed(_scale_table(embedding_weight), idx)
    return out.reshape(_BATCH, _SEQ, _EMBED)


# speedup vs baseline: 1.1631x; 1.1631x over previous
import functools
import math

import jax
import jax.numpy as jnp
from jax import lax
from jax.experimental import pallas as pl
from jax.experimental.pallas import tpu as pltpu
from jax.experimental.pallas import tpu_sc as plsc

_VOCAB = 100000
_EMBED = 128
_BATCH = 4096
_SEQ = 200
_SCALE = math.sqrt(float(_EMBED))

_NC = 2
_NS = 16
_NW = _NC * _NS
_TOKENS = _BATCH * _SEQ
_PER_W = _TOKENS // _NW
_C = 128
_NCHUNK = _PER_W // _C
_NBUF = 5


@functools.partial(
    pl.kernel,
    mesh=plsc.VectorSubcoreMesh(core_axis_name="c", subcore_axis_name="s"),
    out_type=jax.ShapeDtypeStruct((_NW, _NCHUNK, _C, _EMBED), jnp.float32),
    scratch_types=[
        pltpu.VMEM((_NCHUNK, _C), jnp.int32),
        pltpu.VMEM((_C, _EMBED), jnp.float32),
        pltpu.VMEM((_C, _EMBED), jnp.float32),
        pltpu.VMEM((_C, _EMBED), jnp.float32),
        pltpu.VMEM((_C, _EMBED), jnp.float32),
        pltpu.VMEM((_C, _EMBED), jnp.float32),
        pltpu.SemaphoreType.DMA,
        pltpu.SemaphoreType.DMA,
        pltpu.SemaphoreType.DMA,
        pltpu.SemaphoreType.DMA,
        pltpu.SemaphoreType.DMA,
        pltpu.SemaphoreType.DMA,
        pltpu.SemaphoreType.DMA,
        pltpu.SemaphoreType.DMA,
        pltpu.SemaphoreType.DMA,
        pltpu.SemaphoreType.DMA,
    ],
)
def _embed(table_hbm, idx_hbm, out_hbm, idx_v,
           r0, r1, r2, r3, r4, g0, g1, g2, g3, g4, s0, s1, s2, s3, s4):
    wid = lax.axis_index("s") * _NC + lax.axis_index("c")
    pltpu.sync_copy(idx_hbm.at[wid], idx_v)

    bufs = (r0, r1, r2, r3, r4)
    gsems = (g0, g1, g2, g3, g4)
    ssems = (s0, s1, s2, s3, s4)

    def start_gather(g, b):
        pltpu.make_async_copy(table_hbm.at[idx_v.at[g]], bufs[b], gsems[b]).start()

    def wait_gather(b):
        pltpu.make_async_copy(table_hbm.at[idx_v.at[0]], bufs[b], gsems[b]).wait()

    def start_scatter(g, b):
        pltpu.make_async_copy(bufs[b], out_hbm.at[wid, g], ssems[b]).start()

    def wait_scatter(b):
        pltpu.make_async_copy(bufs[b], out_hbm.at[wid, 0], ssems[b]).wait()

    def scale_buf(b):
        buf = bufs[b]

        def row(r, carry):
            for c in range(8):
                sl = pl.ds(16 * c, 16)
                buf[r, sl] = buf[r, sl] * _SCALE
            return carry

        lax.fori_loop(0, _C, row, 0, unroll=4)


    start_gather(0, 0)
    start_gather(1, 1)
    for g in range(5):
        wait_gather(g)
        scale_buf(g)
        start_scatter(g, g)
        b2 = (g + 2) % _NBUF
        if g >= 3:
            wait_scatter(b2)
        start_gather(g + 2, b2)

    def step(r, carry):
        for b in range(_NBUF):
            g = _NBUF * r + b
            wait_gather(b)
            scale_buf(b)
            start_scatter(g, b)
            b2 = (b + 2) % _NBUF
            wait_scatter(b2)
            start_gather(g + 2, b2)
        return carry

    lax.fori_loop(1, _NCHUNK // _NBUF - 1, step, 0)

    for g in (195, 196, 197):
        b = g % _NBUF
        wait_gather(b)
        scale_buf(b)
        start_scatter(g, b)
        b2 = (b + 2) % _NBUF
        wait_scatter(b2)
        start_gather(g + 2, b2)
    for g in (198, 199):
        b = g % _NBUF
        wait_gather(b)
        scale_buf(b)
        start_scatter(g, b)
    for b in range(_NBUF):
        wait_scatter(b)


def kernel(tokens, embedding_weight):
    idx = tokens.astype(jnp.int32).reshape(_NW, _NCHUNK, _C)
    out = _embed(embedding_weight, idx)
    return out.reshape(_BATCH, _SEQ, _EMBED)


# speedup vs baseline: 1.1635x; 1.0003x over previous
import functools
import math

import jax
import jax.numpy as jnp
from jax import lax
from jax.experimental import pallas as pl
from jax.experimental.pallas import tpu as pltpu
from jax.experimental.pallas import tpu_sc as plsc

_VOCAB = 100000
_EMBED = 128
_BATCH = 4096
_SEQ = 200
_SCALE = math.sqrt(float(_EMBED))

_NC = 2
_NS = 16
_NW = _NC * _NS
_TOKENS = _BATCH * _SEQ
_PER_W = _TOKENS // _NW
_C = 128
_NCHUNK = _PER_W // _C
_NBUF = 4
_NROUND = _NCHUNK // _NBUF


@functools.partial(
    pl.kernel,
    mesh=plsc.VectorSubcoreMesh(core_axis_name="c", subcore_axis_name="s"),
    out_type=jax.ShapeDtypeStruct((_NW, _NCHUNK, _C, _EMBED), jnp.float32),
    scratch_types=[
        pltpu.VMEM((_NCHUNK, _C), jnp.int32),
        pltpu.VMEM((_C, _EMBED), jnp.float32),
        pltpu.VMEM((_C, _EMBED), jnp.float32),
        pltpu.VMEM((_C, _EMBED), jnp.float32),
        pltpu.VMEM((_C, _EMBED), jnp.float32),
        pltpu.SemaphoreType.DMA,
        pltpu.SemaphoreType.DMA,
        pltpu.SemaphoreType.DMA,
        pltpu.SemaphoreType.DMA,
        pltpu.SemaphoreType.DMA,
        pltpu.SemaphoreType.DMA,
        pltpu.SemaphoreType.DMA,
        pltpu.SemaphoreType.DMA,
    ],
)
def _embed(table_hbm, idx_hbm, out_hbm, idx_v,
           r0, r1, r2, r3, g0, g1, g2, g3, s0, s1, s2, s3):
    wid = lax.axis_index("s") * _NC + lax.axis_index("c")
    pltpu.sync_copy(idx_hbm.at[wid], idx_v)

    bufs = (r0, r1, r2, r3)
    gsems = (g0, g1, g2, g3)
    ssems = (s0, s1, s2, s3)

    def start_gather(g, b):
        pltpu.make_async_copy(table_hbm.at[idx_v.at[g]], bufs[b], gsems[b]).start()

    def wait_gather(b):
        pltpu.make_async_copy(table_hbm.at[idx_v.at[0]], bufs[b], gsems[b]).wait()

    def start_scatter(g, b):
        pltpu.make_async_copy(bufs[b], out_hbm.at[wid, g], ssems[b]).start()

    def wait_scatter(b):
        pltpu.make_async_copy(bufs[b], out_hbm.at[wid, 0], ssems[b]).wait()

    def scale_buf(b):
        buf = bufs[b]

        def row(r, carry):
            for c in range(8):
                sl = pl.ds(16 * c, 16)
                buf[r, sl] = buf[r, sl] * _SCALE
            return carry

        lax.fori_loop(0, _C, row, 0, unroll=4)


    start_gather(0, 0)
    start_gather(1, 1)
    for b in (0, 1):
        wait_gather(b)
        scale_buf(b)
        start_scatter(b, b)
        start_gather(b + 2, b + 2)
    for b in (2, 3):
        wait_gather(b)
        scale_buf(b)
        start_scatter(b, b)
        wait_scatter(b - 2)
        start_gather(4 + b - 2, b - 2)

    def step(i, carry):
        for b in range(_NBUF):
            wait_gather(b)
            scale_buf(b)
            start_scatter(_NBUF * (i - 1) + b, b)
            b2 = (b + 2) % _NBUF
            wait_scatter(b2)
            if b2 >= 2:
                start_gather(_NBUF * (i - 1) + b2, b2)
            else:
                start_gather(_NBUF * i + b2, b2)
        return carry

    lax.fori_loop(2, _NROUND, step, 0)

    for b in (0, 1):
        wait_gather(b)
        scale_buf(b)
        start_scatter(_NCHUNK - _NBUF + b, b)
        wait_scatter(b + 2)
        start_gather(_NCHUNK - 2 + b, b + 2)
    for b in (2, 3):
        wait_gather(b)
        scale_buf(b)
        start_scatter(_NCHUNK - 2 + b - 2, b)
    for b in range(_NBUF):
        wait_scatter(b)


def kernel(tokens, embedding_weight):
    idx = tokens.astype(jnp.int32).reshape(_NW, _NCHUNK, _C)
    out = _embed(embedding_weight, idx)
    return out.reshape(_BATCH, _SEQ, _EMBED)
